# Initial kernel scaffold; baseline (speedup 1.0000x reference)
import jax
import jax.numpy as jnp
from jax.experimental import pallas as pl


def kernel(h, edge_index, edge_type, W, loop_weight, loop_bias, bias_weight, weight_rel, gating_attention):
    raise NotImplementedError("write your pallas kernel here")


# speedup vs baseline: 7.1556x; 7.1556x over previous
import functools

import jax
import jax.numpy as jnp
from jax import lax
from jax.experimental import pallas as pl
from jax.experimental.pallas import tpu as pltpu
from jax.experimental.pallas import tpu_sc as plsc

_NC = 2
_NS = 16
_K = 80
_CW = 16


@functools.lru_cache(maxsize=None)
def _build(N, E, D, R):
    NW = _NC * _NS
    assert E % (NW * _K) == 0
    CPW = E // (NW * _K)
    ZR = 125
    assert (N // _NS) % ZR == 0
    BN = 1000
    assert N % BN == 0 and N % _NS == 0

    def s1_body(wr_ref, h_ref, w_ref, lw_ref, lb_ref, table_ref, lm_ref):
        wr = wr_ref[...]
        m = jnp.max(wr, axis=0, keepdims=True)
        e = jnp.exp(wr - m)
        a = e / jnp.sum(e, axis=0, keepdims=True)
        h = h_ref[...]
        t = jnp.dot(h, w_ref[...], preferred_element_type=jnp.float32)
        lm_ref[...] = (
            jnp.dot(h, lw_ref[...], preferred_element_type=jnp.float32)
            + lb_ref[...]
        )
        for r in range(R):
            table_ref[r] = t * a[r, 0]

    stage1 = pl.pallas_call(
        s1_body,
        grid=(N // BN,),
        in_specs=[
            pl.BlockSpec((R, 1), lambda i: (0, 0)),
            pl.BlockSpec((BN, D), lambda i: (i, 0)),
            pl.BlockSpec((D, D), lambda i: (0, 0)),
            pl.BlockSpec((D, D), lambda i: (0, 0)),
            pl.BlockSpec((1, D), lambda i: (0, 0)),
        ],
        out_specs=[
            pl.BlockSpec((R, BN, D), lambda i: (0, i, 0)),
            pl.BlockSpec((BN, D), lambda i: (i, 0)),
        ],
        out_shape=[
            jax.ShapeDtypeStruct((R, N, D), jnp.float32),
            jax.ShapeDtypeStruct((N, D), jnp.float32),
        ],
    )

    return stage1, _build_sc(N, E, D, R), _build_combine(N, E, D, R)


@functools.lru_cache(maxsize=None)
def _build_sc(N, E, D, R):
    NW = _NC * _NS
    CPW = E // (NW * _K)
    ZR = 25

    mesh = plsc.VectorSubcoreMesh(core_axis_name="c", subcore_axis_name="s",
                                  num_cores=_NC, num_subcores=_NS)

    @functools.partial(
        pl.kernel,
        out_type=[
            jax.ShapeDtypeStruct((_NC, N, D), jnp.float32),
            jax.ShapeDtypeStruct((_NC, N, _CW), jnp.float32),
        ],
        mesh=mesh,
        scratch_types=[
            pltpu.VMEM((CPW, _K), jnp.int32),
            pltpu.VMEM((CPW, _K), jnp.int32),
            pltpu.VMEM((_K, D), jnp.float32),
            pltpu.VMEM((_K, _CW), jnp.float32),
            pltpu.VMEM((ZR, D), jnp.float32),
            pltpu.VMEM((ZR, _CW), jnp.float32),
            pltpu.VMEM_SHARED((N, D), jnp.float32),
            pltpu.VMEM_SHARED((N, _CW), jnp.float32),
            pltpu.SemaphoreType.DMA,
        ],
        compiler_params=pltpu.CompilerParams(use_tc_tiling_on_sc=False),
    )
    def stage2(pk_hbm, table_hbm, psum_hbm, pcnt_hbm,
               gidx_v, dst_v, rows_v, ones_v, zrow_v, zcnt_v,
               acc, accc, sem):
        c = lax.axis_index("c")
        s = lax.axis_index("s")
        wid = c * _NS + s
        row0 = wid * CPW

        def fill_rows(i, _):
            def fill_cols(j, _):
                zrow_v[i, pl.ds(j * 16, 16)] = jnp.zeros((16,), jnp.float32)
                return 0
            return lax.fori_loop(0, D // 16, fill_cols, 0)
        lax.fori_loop(0, ZR, fill_rows, 0)

        def fill_small(i, _):
            zcnt_v[i, :] = jnp.zeros((_CW,), jnp.float32)
            return 0
        lax.fori_loop(0, ZR, fill_small, 0)

        def fill_ones(i, _):
            ones_v[i, :] = jnp.ones((_CW,), jnp.float32)
            return 0
        lax.fori_loop(0, _K, fill_ones, 0)

        def zero_acc(p, _):
            off = s * (N // _NS) + p * ZR
            pltpu.sync_copy(zrow_v, acc.at[pl.ds(off, ZR)])
            pltpu.sync_copy(zcnt_v, accc.at[pl.ds(off, ZR)])
            return 0
        lax.fori_loop(0, N // _NS // ZR, zero_acc, 0)

        pltpu.sync_copy(pk_hbm.at[pl.ds(row0, CPW)], gidx_v)

        def unpack_row(i, _):
            def unpack_col(j, _):
                sl = pl.ds(j * 16, 16)
                w = gidx_v[i, sl]
                dst_v[i, sl] = w & 0x3FFF
                gidx_v[i, sl] = lax.shift_right_logical(w, 14)
                return 0
            return lax.fori_loop(0, _K // 16, unpack_col, 0)
        lax.fori_loop(0, CPW, unpack_row, 0)

        plsc.subcore_barrier()

        def chunk(i, _):
            pltpu.async_copy(table_hbm.at[gidx_v.at[i]], rows_v, sem).wait()
            pltpu.sync_copy(rows_v, acc.at[dst_v.at[i]], add=True)
            pltpu.sync_copy(ones_v, accc.at[dst_v.at[i]], add=True)
            return 0
        lax.fori_loop(0, CPW, chunk, 0)

        plsc.subcore_barrier()

        roff = s * (N // _NS)
        pltpu.sync_copy(acc.at[pl.ds(roff, N // _NS)],
                        psum_hbm.at[c, pl.ds(roff, N // _NS)])
        pltpu.sync_copy(accc.at[pl.ds(roff, N // _NS)],
                        pcnt_hbm.at[c, pl.ds(roff, N // _NS)])

    return stage2


@functools.lru_cache(maxsize=None)
def _build_combine(N, E, D, R):
    BN = 1000

    def s3_body(psum_ref, pcnt_ref, lm_ref, h_ref, bias_ref, g_ref, out_ref):
        ssum = psum_ref[0] + psum_ref[1]
        cnt = (pcnt_ref[0] + pcnt_ref[1])[:, 0:1]
        mean = ssum / jnp.maximum(cnt, 1.0)
        node = jnp.where(cnt > 0, mean, h_ref[...])
        lm = lm_ref[...]
        logit = jnp.sum(lm * g_ref[0:1, :] + node * g_ref[1:2, :],
                        axis=1, keepdims=True)
        att = jax.nn.sigmoid(logit)
        node = node + bias_ref[...]
        out_ref[...] = node * att + lm * (1.0 - att)

    stage3 = pl.pallas_call(
        s3_body,
        grid=(N // BN,),
        in_specs=[
            pl.BlockSpec((_NC, BN, D), lambda i: (0, i, 0)),
            pl.BlockSpec((_NC, BN, _CW), lambda i: (0, i, 0)),
            pl.BlockSpec((BN, D), lambda i: (i, 0)),
            pl.BlockSpec((BN, D), lambda i: (i, 0)),
            pl.BlockSpec((1, D), lambda i: (0, 0)),
            pl.BlockSpec((2, D), lambda i: (0, 0)),
        ],
        out_specs=pl.BlockSpec((BN, D), lambda i: (i, 0)),
        out_shape=jax.ShapeDtypeStruct((N, D), jnp.float32),
    )

    return stage3


def kernel(h, edge_index, edge_type, W, loop_weight, loop_bias, bias_weight,
           weight_rel, gating_attention):
    N, D = h.shape
    E = edge_type.shape[0]
    R = weight_rel.shape[0]
    stage1, stage2, stage3 = _build(N, E, D, R)

    table, loop_msg = stage1(
        weight_rel, h, W, loop_weight, loop_bias.reshape(1, D))

    gidx = (edge_type * N + edge_index[0]).astype(jnp.uint32)
    packed = lax.bitcast_convert_type(
        (gidx << 14) | edge_index[1].astype(jnp.uint32), jnp.int32)
    psum, pcnt = stage2(packed.reshape(E // _K, _K), table.reshape(R * N, D))

    return stage3(psum, pcnt, loop_msg, h,
                  bias_weight.reshape(1, D), gating_attention.reshape(2, D))


# speedup vs baseline: 10.5444x; 1.4736x over previous
import functools

import jax
import jax.numpy as jnp
from jax import lax
from jax.experimental import pallas as pl
from jax.experimental.pallas import tpu as pltpu
from jax.experimental.pallas import tpu_sc as plsc

_NC = 2
_NS = 16
_K = 80
_CW = 16


@functools.lru_cache(maxsize=None)
def _build(N, E, D, R):
    NW = _NC * _NS
    assert E % (NW * _K) == 0
    CPW = E // (NW * _K)
    ZR = 125
    assert (N // _NS) % ZR == 0
    BN = 1000
    assert N % BN == 0 and N % _NS == 0

    def s1_body(wr_ref, h_ref, w_ref, lw_ref, lb_ref, table_ref, lm_ref):
        wr = wr_ref[...]
        m = jnp.max(wr, axis=0, keepdims=True)
        e = jnp.exp(wr - m)
        a = e / jnp.sum(e, axis=0, keepdims=True)
        h = h_ref[...]
        t = jnp.dot(h, w_ref[...], preferred_element_type=jnp.float32)
        lm_ref[...] = (
            jnp.dot(h, lw_ref[...], preferred_element_type=jnp.float32)
            + lb_ref[...]
        )
        for r in range(R):
            table_ref[r] = t * a[r, 0]

    stage1 = pl.pallas_call(
        s1_body,
        grid=(N // BN,),
        in_specs=[
            pl.BlockSpec((R, 1), lambda i: (0, 0)),
            pl.BlockSpec((BN, D), lambda i: (i, 0)),
            pl.BlockSpec((D, D), lambda i: (0, 0)),
            pl.BlockSpec((D, D), lambda i: (0, 0)),
            pl.BlockSpec((1, D), lambda i: (0, 0)),
        ],
        out_specs=[
            pl.BlockSpec((R, BN, D), lambda i: (0, i, 0)),
            pl.BlockSpec((BN, D), lambda i: (i, 0)),
        ],
        out_shape=[
            jax.ShapeDtypeStruct((R, N, D), jnp.float32),
            jax.ShapeDtypeStruct((N, D), jnp.float32),
        ],
    )

    return stage1, _build_sc(N, E, D, R), _build_combine(N, E, D, R)


@functools.lru_cache(maxsize=None)
def _build_sc(N, E, D, R):
    NW = _NC * _NS
    CPW = E // (NW * _K)
    assert CPW % 2 == 1
    ZR = 25

    mesh = plsc.VectorSubcoreMesh(core_axis_name="c", subcore_axis_name="s",
                                  num_cores=_NC, num_subcores=_NS)

    @functools.partial(
        pl.kernel,
        out_type=[
            jax.ShapeDtypeStruct((_NC, N, D), jnp.float32),
            jax.ShapeDtypeStruct((_NC, N, _CW), jnp.float32),
        ],
        mesh=mesh,
        scratch_types=[
            pltpu.VMEM((CPW, _K), jnp.int32),
            pltpu.VMEM((_K,), jnp.int32),
            pltpu.VMEM((_K,), jnp.int32),
            pltpu.VMEM((_K,), jnp.int32),
            pltpu.VMEM((_K,), jnp.int32),
            pltpu.VMEM((_K, D), jnp.float32),
            pltpu.VMEM((_K, D), jnp.float32),
            pltpu.VMEM((_K, _CW), jnp.float32),
            pltpu.VMEM((ZR, D), jnp.float32),
            pltpu.VMEM((ZR, _CW), jnp.float32),
            pltpu.VMEM_SHARED((N, D), jnp.float32),
            pltpu.VMEM_SHARED((N, _CW), jnp.float32),
            pltpu.SemaphoreType.DMA,
            pltpu.SemaphoreType.DMA,
            pltpu.SemaphoreType.DMA,
            pltpu.SemaphoreType.DMA,
            pltpu.SemaphoreType.DMA,
            pltpu.SemaphoreType.DMA,
            pltpu.SemaphoreType.DMA,
        ],
        compiler_params=pltpu.CompilerParams(use_tc_tiling_on_sc=False),
    )
    def stage2(pk_hbm, table_hbm, psum_hbm, pcnt_hbm,
               pk_v, gia, dsa, gib, dsb, rowsa, rowsb, ones_v, zrow_v, zcnt_v,
               acc, accc, g_a, g_b, s_a, s_b, c_a, c_b, z_sem):
        c = lax.axis_index("c")
        s = lax.axis_index("s")
        wid = c * _NS + s
        row0 = wid * CPW

        def fill_rows(i, _):
            def fill_cols(j, _):
                zrow_v[i, pl.ds(j * 16, 16)] = jnp.zeros((16,), jnp.float32)
                return 0
            return lax.fori_loop(0, D // 16, fill_cols, 0)
        lax.fori_loop(0, ZR, fill_rows, 0)

        def fill_small(i, _):
            zcnt_v[i, :] = jnp.zeros((_CW,), jnp.float32)
            ones_v[i, :] = jnp.ones((_CW,), jnp.float32)
            return 0
        lax.fori_loop(0, ZR, fill_small, 0)

        def fill_ones(i, _):
            ones_v[ZR + i, :] = jnp.ones((_CW,), jnp.float32)
            return 0
        lax.fori_loop(0, _K - ZR, fill_ones, 0)

        nz = N // _NS // ZR

        def zero_acc(p, _):
            off = s * (N // _NS) + p * ZR
            pltpu.async_copy(zrow_v, acc.at[pl.ds(off, ZR)], z_sem)
            pltpu.async_copy(zcnt_v, accc.at[pl.ds(off, ZR)], z_sem)
            return 0
        lax.fori_loop(0, nz, zero_acc, 0)

        pltpu.sync_copy(pk_hbm.at[pl.ds(row0, CPW)], pk_v)

        def drain_zero(p, _):
            off = s * (N // _NS) + p * ZR
            pltpu.make_async_copy(zrow_v, acc.at[pl.ds(off, ZR)], z_sem).wait()
            pltpu.make_async_copy(zcnt_v, accc.at[pl.ds(off, ZR)], z_sem).wait()
            return 0
        lax.fori_loop(0, nz, drain_zero, 0)

        def unpack_into(i, gb, db):
            def col(j, _):
                sl = pl.ds(j * 16, 16)
                w = pk_v[i, sl]
                db[sl] = w & 0x3FFF
                gb[sl] = lax.shift_right_logical(w, 14)
                return 0
            lax.fori_loop(0, _K // 16, col, 0)

        plsc.subcore_barrier()

        unpack_into(0, gia, dsa)
        pltpu.async_copy(table_hbm.at[gia], rowsa, g_a)

        def pair(t, _):
            j = 2 * t

            @pl.when(t > 0)
            def _():
                pltpu.make_async_copy(rowsb, acc.at[dsb], s_b).wait()
                pltpu.make_async_copy(ones_v, accc.at[dsb], c_b).wait()

            unpack_into(j + 1, gib, dsb)
            pltpu.async_copy(table_hbm.at[gib], rowsb, g_b)
            pltpu.make_async_copy(table_hbm.at[gia], rowsa, g_a).wait()
            pltpu.async_copy(rowsa, acc.at[dsa], s_a, add=True)
            pltpu.async_copy(ones_v, accc.at[dsa], c_a, add=True)
            pltpu.make_async_copy(rowsa, acc.at[dsa], s_a).wait()
            pltpu.make_async_copy(ones_v, accc.at[dsa], c_a).wait()
            unpack_into(j + 2, gia, dsa)
            pltpu.async_copy(table_hbm.at[gia], rowsa, g_a)
            pltpu.make_async_copy(table_hbm.at[gib], rowsb, g_b).wait()
            pltpu.async_copy(rowsb, acc.at[dsb], s_b, add=True)
            pltpu.async_copy(ones_v, accc.at[dsb], c_b, add=True)
            return 0
        lax.fori_loop(0, (CPW - 1) // 2, pair, 0)

        pltpu.make_async_copy(table_hbm.at[gia], rowsa, g_a).wait()
        pltpu.async_copy(rowsa, acc.at[dsa], s_a, add=True)
        pltpu.async_copy(ones_v, accc.at[dsa], c_a, add=True)
        pltpu.make_async_copy(rowsb, acc.at[dsb], s_b).wait()
        pltpu.make_async_copy(ones_v, accc.at[dsb], c_b).wait()
        pltpu.make_async_copy(rowsa, acc.at[dsa], s_a).wait()
        pltpu.make_async_copy(ones_v, accc.at[dsa], c_a).wait()

        plsc.subcore_barrier()

        roff = s * (N // _NS)
        pltpu.sync_copy(acc.at[pl.ds(roff, N // _NS)],
                        psum_hbm.at[c, pl.ds(roff, N // _NS)])
        pltpu.sync_copy(accc.at[pl.ds(roff, N // _NS)],
                        pcnt_hbm.at[c, pl.ds(roff, N // _NS)])

    return stage2


@functools.lru_cache(maxsize=None)
def _build_combine(N, E, D, R):
    BN = 1000

    def s3_body(psum_ref, pcnt_ref, lm_ref, h_ref, bias_ref, g_ref, out_ref):
        ssum = psum_ref[0] + psum_ref[1]
        cnt = (pcnt_ref[0] + pcnt_ref[1])[:, 0:1]
        mean = ssum / jnp.maximum(cnt, 1.0)
        node = jnp.where(cnt > 0, mean, h_ref[...])
        lm = lm_ref[...]
        logit = jnp.sum(lm * g_ref[0:1, :] + node * g_ref[1:2, :],
                        axis=1, keepdims=True)
        att = jax.nn.sigmoid(logit)
        node = node + bias_ref[...]
        out_ref[...] = node * att + lm * (1.0 - att)

    stage3 = pl.pallas_call(
        s3_body,
        grid=(N // BN,),
        in_specs=[
            pl.BlockSpec((_NC, BN, D), lambda i: (0, i, 0)),
            pl.BlockSpec((_NC, BN, _CW), lambda i: (0, i, 0)),
            pl.BlockSpec((BN, D), lambda i: (i, 0)),
            pl.BlockSpec((BN, D), lambda i: (i, 0)),
            pl.BlockSpec((1, D), lambda i: (0, 0)),
            pl.BlockSpec((2, D), lambda i: (0, 0)),
        ],
        out_specs=pl.BlockSpec((BN, D), lambda i: (i, 0)),
        out_shape=jax.ShapeDtypeStruct((N, D), jnp.float32),
    )

    return stage3


def kernel(h, edge_index, edge_type, W, loop_weight, loop_bias, bias_weight,
           weight_rel, gating_attention):
    N, D = h.shape
    E = edge_type.shape[0]
    R = weight_rel.shape[0]
    stage1, stage2, stage3 = _build(N, E, D, R)

    table, loop_msg = stage1(
        weight_rel, h, W, loop_weight, loop_bias.reshape(1, D))

    gidx = (edge_type * N + edge_index[0]).astype(jnp.uint32)
    packed = lax.bitcast_convert_type(
        (gidx << 14) | edge_index[1].astype(jnp.uint32), jnp.int32)
    psum, pcnt = stage2(packed.reshape(E // _K, _K), table.reshape(R * N, D))

    return stage3(psum, pcnt, loop_msg, h,
                  bias_weight.reshape(1, D), gating_attention.reshape(2, D))
